# baseline (speedup 1.0000x reference)
import jax
import jax.numpy as jnp
from jax.experimental import pallas as pl
from jax.experimental.pallas import tpu as pltpu

INPUT_SIZE = 256
OUT_SIZE = 256
MEMORY_FEATURE = 128

_NBUF = 8
_CHUNK = 4096
_LEAD = 4


def _make_body(batch, memory_size):
    nchunks = memory_size // _CHUNK
    nslabs = 4
    bm = batch // nslabs
    slab_iters = {1 + 3 * j: j for j in range(nslabs)}

    def _body(x_hbm, mem_hbm, w_ref, b_ref, out_hbm, mstate_hbm,
              bufs, x_buf, out_buf, mslice, in_sems, out_sems,
              x_sem, o_sem):
        x_dma = pltpu.make_async_copy(x_hbm, x_buf, x_sem)
        x_dma.start()
        in_dmas = [None] * nchunks
        out_dmas = [None] * nchunks
        for c in range(_LEAD):
            in_dmas[c] = pltpu.make_async_copy(
                mem_hbm.at[pl.ds(c * _CHUNK, _CHUNK)],
                bufs.at[c % _NBUF], in_sems.at[c % _NBUF])
            in_dmas[c].start()

        out_dma_started = False
        for k in range(nchunks):
            nc = k + _LEAD
            if nc < nchunks:
                nb = nc % _NBUF
                if k >= _LEAD:
                    out_dmas[nc - _NBUF].wait()
                in_dmas[nc] = pltpu.make_async_copy(
                    mem_hbm.at[pl.ds(nc * _CHUNK, _CHUNK)],
                    bufs.at[nb], in_sems.at[nb])
                in_dmas[nc].start()
            b = k % _NBUF
            in_dmas[k].wait()
            if k == 0:
                mslice[...] = bufs[0, :batch, :]
            out_dmas[k] = pltpu.make_async_copy(
                bufs.at[b], mstate_hbm.at[pl.ds(k * _CHUNK, _CHUNK)],
                out_sems.at[b])
            out_dmas[k].start()
            if k in slab_iters:
                j = slab_iters[k]
                if j == 0:
                    x_dma.wait()
                acc = jnp.dot(x_buf[pl.ds(j * bm, bm), :],
                              w_ref[:INPUT_SIZE, :],
                              preferred_element_type=jnp.float32)
                acc = acc + jnp.dot(mslice[pl.ds(j * bm, bm), :],
                                    w_ref[INPUT_SIZE:, :],
                                    preferred_element_type=jnp.float32)
                out_buf[pl.ds(j * bm, bm), :] = acc + b_ref[...]
                if j == nslabs - 1:
                    pltpu.make_async_copy(out_buf, out_hbm, o_sem).start()
                    out_dma_started = True
        for c in range(nchunks - _NBUF, nchunks):
            out_dmas[c].wait()
        if out_dma_started:
            pltpu.make_async_copy(out_buf, out_hbm, o_sem).wait()

    return _body


def kernel(x, mem, W, b):
    batch, _ = x.shape
    memory_size = mem.shape[0]
    b2 = b.reshape(1, OUT_SIZE)
    out, mem_state = pl.pallas_call(
        _make_body(batch, memory_size),
        in_specs=[
            pl.BlockSpec(memory_space=pltpu.MemorySpace.HBM),
            pl.BlockSpec(memory_space=pltpu.MemorySpace.HBM),
            pl.BlockSpec((INPUT_SIZE + MEMORY_FEATURE, OUT_SIZE),
                         lambda: (0, 0)),
            pl.BlockSpec((1, OUT_SIZE), lambda: (0, 0)),
        ],
        out_specs=[
            pl.BlockSpec(memory_space=pltpu.MemorySpace.HBM),
            pl.BlockSpec(memory_space=pltpu.MemorySpace.HBM),
        ],
        out_shape=[
            jax.ShapeDtypeStruct((batch, OUT_SIZE), jnp.float32),
            jax.ShapeDtypeStruct(mem.shape, mem.dtype),
        ],
        scratch_shapes=[
            pltpu.VMEM((_NBUF, _CHUNK, MEMORY_FEATURE), jnp.float32),
            pltpu.VMEM((batch, INPUT_SIZE), jnp.float32),
            pltpu.VMEM((batch, OUT_SIZE), jnp.float32),
            pltpu.VMEM((batch, MEMORY_FEATURE), jnp.float32),
            pltpu.SemaphoreType.DMA((_NBUF,)),
            pltpu.SemaphoreType.DMA((_NBUF,)),
            pltpu.SemaphoreType.DMA,
            pltpu.SemaphoreType.DMA,
        ],
    )(x, mem, W, b2)
    return (out, mem_state)
